# baseline (device time: 21207 ns/iter reference)
import jax
import jax.numpy as jnp
from jax import lax
from jax.experimental import pallas as pl
from jax.experimental.pallas import tpu as pltpu

N_DEV = 16
BLOCK_M = 512
NC = 4


def kernel(x):
    m_per, n = x.shape
    nr = m_per // BLOCK_M
    cw = n // NC

    def body(x_ref, out_ref, gather_ref, send_sems, recv_sems):
        my_pos = lax.axis_index("i")
        c = pl.program_id(0)
        r = pl.program_id(1)

        barrier_sem = pltpu.get_barrier_semaphore()

        @pl.when((c == 0) & (r == 0))
        def _signal():
            for d in range(1, N_DEV):
                pl.semaphore_signal(
                    barrier_sem,
                    inc=1,
                    device_id=((my_pos + d) % N_DEV,),
                    device_id_type=pl.DeviceIdType.MESH,
                )

        xv = x_ref[:, :]
        bmax = jnp.max(xv, axis=0)
        bidx = (
            my_pos * m_per + r * BLOCK_M + jnp.argmax(xv, axis=0)
        ).astype(jnp.float32)
        cols = pl.ds(c * cw, cw)

        @pl.when(r == 0)
        def _init():
            gather_ref[0, 0, cols] = bmax
            gather_ref[0, 1, cols] = bidx

        @pl.when(r > 0)
        def _combine():
            run_v = gather_ref[0, 0, cols]
            better = bmax > run_v
            gather_ref[0, 0, cols] = jnp.where(better, bmax, run_v)
            gather_ref[0, 1, cols] = jnp.where(
                better, bidx, gather_ref[0, 1, cols]
            )

        @pl.when(r == nr - 1)
        def _send():
            @pl.when(c == 0)
            def _barrier_wait():
                pl.semaphore_wait(barrier_sem, N_DEV - 1)

            for cc in range(NC):

                @pl.when(c == cc)
                def _(cc=cc):
                    ccols = pl.ds(cc * cw, cw)
                    for d in range(1, N_DEV):
                        pltpu.make_async_remote_copy(
                            src_ref=gather_ref.at[0, :, ccols],
                            dst_ref=gather_ref.at[d, :, ccols],
                            send_sem=send_sems.at[cc, d],
                            recv_sem=recv_sems.at[cc, d],
                            device_id=((my_pos + d) % N_DEV,),
                            device_id_type=pl.DeviceIdType.MESH,
                        ).start()

        @pl.when((c == NC - 1) & (r == nr - 1))
        def _finish():
            for cc in range(NC):
                ccols = pl.ds(cc * cw, cw)
                waits = []
                for d in range(1, N_DEV):
                    rdma = pltpu.make_async_remote_copy(
                        src_ref=gather_ref.at[0, :, ccols],
                        dst_ref=gather_ref.at[d, :, ccols],
                        send_sem=send_sems.at[cc, d],
                        recv_sem=recv_sems.at[cc, d],
                        device_id=((my_pos + 1) % N_DEV,),
                        device_id_type=pl.DeviceIdType.MESH,
                    )
                    rdma.wait_recv()
                    waits.append(rdma)

                vals = gather_ref[:, 0, ccols]
                idxs = gather_ref[:, 1, ccols]
                gv = jnp.max(vals, axis=0)
                out_ref[0, ccols] = gv
                out_ref[1, ccols] = jnp.min(
                    jnp.where(vals == gv[None, :], idxs, jnp.float32(2.0**30)),
                    axis=0,
                )
                for rdma in waits:
                    rdma.wait_send()

    return pl.pallas_call(
        body,
        grid=(NC, nr),
        out_shape=jax.ShapeDtypeStruct((2, n), jnp.float32),
        in_specs=[
            pl.BlockSpec(
                (BLOCK_M, cw), lambda c, r: (r, c), memory_space=pltpu.VMEM
            )
        ],
        out_specs=pl.BlockSpec(
            (2, n), lambda c, r: (0, 0), memory_space=pltpu.VMEM
        ),
        scratch_shapes=[
            pltpu.VMEM((N_DEV, 2, n), jnp.float32),
            pltpu.SemaphoreType.DMA((NC, N_DEV)),
            pltpu.SemaphoreType.DMA((NC, N_DEV)),
        ],
        compiler_params=pltpu.CompilerParams(
            collective_id=0, dimension_semantics=("arbitrary", "arbitrary")
        ),
    )(x)


# device time: 17472 ns/iter; 1.2138x vs baseline; 1.2138x over previous
import jax
import jax.numpy as jnp
from jax import lax
from jax.experimental import pallas as pl
from jax.experimental.pallas import tpu as pltpu

N_DEV = 16
BLOCK_M = 512
NS = 2


def kernel(x):
    m_per, n = x.shape
    nr = m_per // BLOCK_M
    rps = nr // NS

    def body(x_ref, out_ref, gather_ref, send_sems, recv_sems):
        my_pos = lax.axis_index("i")
        r = pl.program_id(0)

        barrier_sem = pltpu.get_barrier_semaphore()

        @pl.when(r == 0)
        def _signal():
            for d in range(1, N_DEV):
                pl.semaphore_signal(
                    barrier_sem,
                    inc=1,
                    device_id=((my_pos + d) % N_DEV,),
                    device_id_type=pl.DeviceIdType.MESH,
                )

        xv = x_ref[:, :]
        bmax = jnp.max(xv, axis=0)
        bidx = (
            my_pos * m_per + r * BLOCK_M + jnp.argmax(xv, axis=0)
        ).astype(jnp.float32)
        seg = r // rps

        @pl.when(r % rps == 0)
        def _init():
            gather_ref[seg, 0, 0, :] = bmax
            gather_ref[seg, 0, 1, :] = bidx

        @pl.when(r % rps > 0)
        def _combine():
            run_v = gather_ref[seg, 0, 0, :]
            better = bmax > run_v
            gather_ref[seg, 0, 0, :] = jnp.where(better, bmax, run_v)
            gather_ref[seg, 0, 1, :] = jnp.where(
                better, bidx, gather_ref[seg, 0, 1, :]
            )

        @pl.when(r % rps == rps - 1)
        def _send():
            @pl.when(r == rps - 1)
            def _barrier_wait():
                pl.semaphore_wait(barrier_sem, N_DEV - 1)

            for ss in range(NS):

                @pl.when(seg == ss)
                def _(ss=ss):
                    for d in range(1, N_DEV):
                        pltpu.make_async_remote_copy(
                            src_ref=gather_ref.at[ss, 0],
                            dst_ref=gather_ref.at[ss, d],
                            send_sem=send_sems.at[ss, d],
                            recv_sem=recv_sems.at[ss, d],
                            device_id=((my_pos + d) % N_DEV,),
                            device_id_type=pl.DeviceIdType.MESH,
                        ).start()

        @pl.when(r == nr - 1)
        def _finish():
            rdmas = []
            for ss in range(NS):
                for d in range(1, N_DEV):
                    rdma = pltpu.make_async_remote_copy(
                        src_ref=gather_ref.at[ss, 0],
                        dst_ref=gather_ref.at[ss, d],
                        send_sem=send_sems.at[ss, d],
                        recv_sem=recv_sems.at[ss, d],
                        device_id=((my_pos + 1) % N_DEV,),
                        device_id_type=pl.DeviceIdType.MESH,
                    )
                    rdma.wait_recv()
                    rdmas.append(rdma)

            vals = gather_ref[:, :, 0, :].reshape(NS * N_DEV, n)
            idxs = gather_ref[:, :, 1, :].reshape(NS * N_DEV, n)
            gv = jnp.max(vals, axis=0)
            out_ref[0, :] = gv
            out_ref[1, :] = jnp.min(
                jnp.where(vals == gv[None, :], idxs, jnp.float32(2.0**30)),
                axis=0,
            )
            for rdma in rdmas:
                rdma.wait_send()

    return pl.pallas_call(
        body,
        grid=(nr,),
        out_shape=jax.ShapeDtypeStruct((2, n), jnp.float32),
        in_specs=[
            pl.BlockSpec((BLOCK_M, n), lambda r: (r, 0), memory_space=pltpu.VMEM)
        ],
        out_specs=pl.BlockSpec((2, n), lambda r: (0, 0), memory_space=pltpu.VMEM),
        scratch_shapes=[
            pltpu.VMEM((NS, N_DEV, 2, n), jnp.float32),
            pltpu.SemaphoreType.DMA((NS, N_DEV)),
            pltpu.SemaphoreType.DMA((NS, N_DEV)),
        ],
        compiler_params=pltpu.CompilerParams(
            collective_id=0, dimension_semantics=("arbitrary",)
        ),
    )(x)
